# baseline (speedup 1.0000x reference)
import functools

import jax
import jax.numpy as jnp
from jax import lax
from jax.experimental import pallas as pl
from jax.experimental.pallas import tpu as pltpu
from jax.experimental.pallas import tpu_sc as plsc

_BATCH = 4096
_DIM = 128
_NC = 2
_NS = 16
_NW = _NC * _NS
_ROWS_PER_W = _BATCH // _NW

_MESH = plsc.VectorSubcoreMesh(core_axis_name="c", subcore_axis_name="s")


@functools.partial(
    pl.kernel,
    mesh=_MESH,
    out_type=jax.ShapeDtypeStruct((_BATCH, _DIM), jnp.float32),
)
def _sc_copy(in_hbm, out_hbm):
    wid = lax.axis_index("s") * _NC + lax.axis_index("c")
    base = wid * _ROWS_PER_W
    pltpu.sync_copy(in_hbm.at[pl.ds(base, _ROWS_PER_W)],
                    out_hbm.at[pl.ds(base, _ROWS_PER_W)])


def kernel(initial_space, finite_space, time_embedding):
    return _sc_copy(initial_space)


# speedup vs baseline: 1.0241x; 1.0241x over previous
import functools

import jax
import jax.numpy as jnp
from jax import lax
from jax.experimental import pallas as pl
from jax.experimental.pallas import tpu as pltpu
from jax.experimental.pallas import tpu_sc as plsc

_BATCH = 4096
_DIM = 128
_NC = 2
_ROWS_PER_C = _BATCH // _NC

_MESH = plsc.ScalarSubcoreMesh(axis_name="c", num_cores=_NC)


@functools.partial(
    pl.kernel,
    mesh=_MESH,
    out_type=jax.ShapeDtypeStruct((_BATCH, _DIM), jnp.float32),
)
def _sc_copy(in_hbm, out_hbm):
    cid = lax.axis_index("c")
    base = cid * _ROWS_PER_C
    pltpu.sync_copy(in_hbm.at[pl.ds(base, _ROWS_PER_C)],
                    out_hbm.at[pl.ds(base, _ROWS_PER_C)])


def kernel(initial_space, finite_space, time_embedding):
    return _sc_copy(initial_space)


# speedup vs baseline: 13.9361x; 13.6079x over previous
import jax
import jax.numpy as jnp
from jax.experimental import pallas as pl
from jax.experimental.pallas import tpu as pltpu

_BATCH = 4096
_DIM = 128
_NBLK = 8
_ROWS = _BATCH // _NBLK


def _copy_body(x_ref, o_ref):
    o_ref[...] = x_ref[...]


def kernel(initial_space, finite_space, time_embedding):
    return pl.pallas_call(
        _copy_body,
        grid=(_NBLK,),
        in_specs=[pl.BlockSpec((_ROWS, _DIM), lambda i: (i, 0))],
        out_specs=pl.BlockSpec((_ROWS, _DIM), lambda i: (i, 0)),
        out_shape=jax.ShapeDtypeStruct((_BATCH, _DIM), jnp.float32),
        compiler_params=pltpu.CompilerParams(
            dimension_semantics=("arbitrary",),
        ),
    )(initial_space)
